# baseline (speedup 1.0000x reference)
import functools

import jax
import jax.numpy as jnp
from jax import lax
from jax.experimental import pallas as pl
from jax.experimental.pallas import tpu as pltpu
from jax.experimental.pallas import tpu_sc as plsc

_B_TC = 8192


def _transform_body(table_ref, w_ref, b_ref, gamma_ref, beta_ref,
                    out_ref, out_bf_ref):
    x = table_ref[...]
    h = lax.dot_general(
        x, w_ref[...], (((1,), (1,)), ((), ())),
        preferred_element_type=jnp.float32,
    )
    h = h + b_ref[...]
    mean = jnp.mean(h, axis=1, keepdims=True)
    hc = h - mean
    var = jnp.mean(hc * hc, axis=1, keepdims=True)
    h = hc * lax.rsqrt(var + 1e-5) * gamma_ref[...] + beta_ref[...]
    norm2 = jnp.sum(h * h, axis=1, keepdims=True)
    y = h * lax.rsqrt(jnp.maximum(norm2, 1e-24))
    out_ref[...] = y
    out_bf_ref[...] = y.astype(jnp.bfloat16)


def _transform_table(table, W, b, gamma, beta):
    n, d = table.shape
    return pl.pallas_call(
        _transform_body,
        out_shape=(
            jax.ShapeDtypeStruct((n, d), jnp.float32),
            jax.ShapeDtypeStruct((n, d), jnp.bfloat16),
        ),
    )(table, W, b.reshape(1, d), gamma.reshape(1, d), beta.reshape(1, d))


def _make_sc_gather(b_total, b_sc, d):
    info = plsc.get_sparse_core_info()
    nw = info.num_cores * info.num_subcores
    b_per_w = b_sc // nw
    chunk = 64
    n_chunks = b_per_w // chunk
    out_base = b_total - b_sc
    mesh = plsc.VectorSubcoreMesh(core_axis_name="c", subcore_axis_name="s")

    @functools.partial(
        pl.kernel,
        out_type=jax.ShapeDtypeStruct((b_total, d), jnp.float32),
        mesh=mesh,
        scratch_types=[
            pltpu.VMEM((b_per_w,), jnp.int32),
            pltpu.VMEM((chunk, d), jnp.float32),
            pltpu.VMEM((chunk, d), jnp.float32),
            pltpu.SemaphoreType.DMA,
            pltpu.SemaphoreType.DMA,
            pltpu.SemaphoreType.DMA,
            pltpu.SemaphoreType.DMA,
        ],
    )
    def gather_k(tab_hbm, idx_hbm, out_hbm, idx_v, rows0, rows1,
                 gsem0, gsem1, ssem0, ssem1):
        wid = lax.axis_index("s") * info.num_cores + lax.axis_index("c")
        base = wid * b_per_w
        bufs = (rows0, rows1)
        gsems = (gsem0, gsem1)
        ssems = (ssem0, ssem1)
        pltpu.sync_copy(idx_hbm.at[pl.ds(base, b_per_w)], idx_v)
        gat = [None, None]
        sto = [None, None]
        for c in range(n_chunks + 1):
            if c < n_chunks:
                i = c % 2
                if sto[i] is not None:
                    sto[i].wait()
                gat[i] = pltpu.async_copy(
                    tab_hbm.at[idx_v.at[pl.ds(c * chunk, chunk)]],
                    bufs[i], gsems[i])
            if c >= 1:
                j = (c - 1) % 2
                gat[j].wait()
                sto[j] = pltpu.async_copy(
                    bufs[j],
                    out_hbm.at[pl.ds(out_base + base + (c - 1) * chunk, chunk)],
                    ssems[j])
        for s in sto:
            if s is not None:
                s.wait()

    return gather_k


def _onehot_body(ids_ref, ytab_ref, _aliased_ref, out_ref):
    n_vocab = ytab_ref.shape[0]
    ids = ids_ref[0, 0, :]
    col = lax.broadcasted_iota(jnp.int32, (ids.shape[0], n_vocab), 1)
    oh = (ids[:, None] == col).astype(jnp.bfloat16)
    out_ref[...] = jnp.dot(oh, ytab_ref[...],
                           preferred_element_type=jnp.float32)


def _tc_gather(ids_tc, y_bf, sc_out):
    b_tc = ids_tc.shape[0]
    b_total, d = sc_out.shape
    n_vocab = y_bf.shape[0]
    blk = 512
    grid = (b_tc // blk,)
    ids3 = ids_tc.reshape(grid[0], 1, blk)
    return pl.pallas_call(
        _onehot_body,
        grid=grid,
        in_specs=[
            pl.BlockSpec((1, 1, blk), lambda i: (i, 0, 0)),
            pl.BlockSpec((n_vocab, d), lambda i: (0, 0)),
            pl.BlockSpec(memory_space=pl.ANY),
        ],
        out_specs=pl.BlockSpec((blk, d), lambda i: (i, 0)),
        out_shape=jax.ShapeDtypeStruct((b_total, d), jnp.float32),
        input_output_aliases={2: 0},
    )(ids3, y_bf, sc_out)


def kernel(ids, table, W, b, gamma, beta):
    y_table, y_bf = _transform_table(table, W, b, gamma, beta)
    b_total = ids.shape[0]
    ids32 = ids.astype(jnp.int32)
    sc_gather = _make_sc_gather(b_total, b_total - _B_TC, table.shape[1])
    sc_out = sc_gather(y_table, ids32[_B_TC:])
    return _tc_gather(ids32[:_B_TC], y_bf, sc_out)
